# baseline (speedup 1.0000x reference)
import functools

import jax
import jax.numpy as jnp
from jax import lax
from jax.experimental import pallas as pl
from jax.experimental.pallas import tpu as pltpu
from jax.experimental.pallas import tpu_sc as plsc

S, D = 2048, 768
HQ, HKV, HD = 12, 3, 64
E, TOPK, HEXP = 8, 2, 1536
THETA = 10000.0
EPS = 1e-6
HP = 128
DKP = HKV * HP
RB = 256
BLK = 128
R = S * TOPK + E * BLK
NB = R // BLK
NEG = -1e30


def _rope(t, c, s):
    lane = jax.lax.broadcasted_iota(jnp.int32, t.shape, 1)
    sw = jnp.where(lane % 2 == 0,
                   jnp.roll(t, -1, axis=1),
                   jnp.roll(t, 1, axis=1))
    return t * c + sw * s


def _stage_a_kern(x_ref, w1_ref, qw_ref, kw_ref, vw_ref,
                  cq_ref, sq_ref, ck_ref, sk_ref,
                  q_ref, k_ref, v_ref):
    xb = x_ref[...]
    ms = jnp.mean(xb * xb, axis=1, keepdims=True)
    xn = (xb / jnp.sqrt(ms + EPS)) * w1_ref[...]
    q = jnp.dot(xn, qw_ref[...], preferred_element_type=jnp.float32)
    k = jnp.dot(xn, kw_ref[...], preferred_element_type=jnp.float32)
    v = jnp.dot(xn, vw_ref[...], preferred_element_type=jnp.float32)
    q_ref[...] = _rope(q, cq_ref[...], sq_ref[...])
    k_ref[...] = _rope(k, ck_ref[...], sk_ref[...])
    v_ref[...] = v


def _stage_b_kern(q_ref, k_ref, v_ref, *refs, qb_off, ncols):
    o_ref = refs[-1]
    qb = pl.program_id(1) + qb_off
    kh = k_ref[:, :HD]
    vh = v_ref[:, :HD]
    row = jax.lax.broadcasted_iota(jnp.int32, (RB, ncols), 0) + qb * RB
    col = jax.lax.broadcasted_iota(jnp.int32, (RB, ncols), 1)
    mask = row >= col
    ctxs = []
    for j in range(4):
        qh = q_ref[:, j * HD:(j + 1) * HD]
        sc = jax.lax.dot_general(qh, kh, (((1,), (1,)), ((), ())),
                                 preferred_element_type=jnp.float32)
        sc = jnp.where(mask, sc, NEG)
        m = jnp.max(sc, axis=1, keepdims=True)
        p = jnp.exp(sc - m)
        p = jnp.where(mask, p, 0.0)
        p = p / jnp.sum(p, axis=1, keepdims=True)
        ctxs.append(jnp.dot(p, vh, preferred_element_type=jnp.float32))
    o_ref[:, 0:2 * HD] = jnp.concatenate(ctxs[0:2], axis=1)
    o_ref[:, 2 * HD:4 * HD] = jnp.concatenate(ctxs[2:4], axis=1)


def _stage_c_kern(ctx_ref, x_ref, ow_ref, w2_ref, gw_ref,
                  h_ref, h2_ref, ti_ref, tw_ref):
    h = x_ref[...] + jnp.dot(ctx_ref[...], ow_ref[...],
                             preferred_element_type=jnp.float32)
    h_ref[...] = h
    ms = jnp.mean(h * h, axis=1, keepdims=True)
    h2 = (h / jnp.sqrt(ms + EPS)) * w2_ref[...]
    h2_ref[...] = h2
    logits = jnp.dot(h2, gw_ref[...], preferred_element_type=jnp.float32)
    lane = jax.lax.broadcasted_iota(jnp.int32, (RB, E), 1)
    m1 = jnp.max(logits, axis=1, keepdims=True)
    i1 = jnp.min(jnp.where(logits == m1, lane, E), axis=1, keepdims=True)
    l2 = jnp.where(lane == i1, NEG, logits)
    m2 = jnp.max(l2, axis=1, keepdims=True)
    i2 = jnp.min(jnp.where(l2 == m2, lane, E), axis=1, keepdims=True)
    w1 = 1.0 / (1.0 + jnp.exp(m2 - m1))
    ti_ref[...] = jnp.where(lane == 0, i1, jnp.where(lane == 1, i2, 0))
    tw_ref[...] = jnp.where(lane == 0, w1, jnp.where(lane == 1, 1.0 - w1, 0.0))


def _stage_d_kern(be_ref, bl_ref, xs_ref, wgu_ref, wd_ref, y_ref):
    @pl.when(bl_ref[pl.program_id(0)] != 0)
    def _():
        gu = jnp.dot(xs_ref[...], wgu_ref[0],
                     preferred_element_type=jnp.float32)
        g = gu[:, :HEXP]
        u = gu[:, HEXP:]
        act = (g / (1.0 + jnp.exp(-g))) * u
        y_ref[...] = jnp.dot(act, wd_ref[0],
                             preferred_element_type=jnp.float32)


def _stage_e_kern(h_ref, tw_ref, y0_ref, y1_ref, o_ref):
    w0 = tw_ref[:, 0:1]
    w1 = tw_ref[:, 1:2]
    o_ref[...] = h_ref[...] + w0 * y0_ref[...] + w1 * y1_ref[...]


def _sc_gather(table, idx, chunk):
    bn = idx.shape[0]
    dd = table.shape[1]
    info = plsc.get_sparse_core_info()
    nc, ns = info.num_cores, info.num_subcores
    nw = nc * ns
    bpw = bn // nw
    nch = bpw // chunk
    idx3 = idx.reshape(nw, nch, chunk)

    @functools.partial(
        pl.kernel,
        mesh=plsc.VectorSubcoreMesh(core_axis_name="c", subcore_axis_name="s"),
        out_type=jax.ShapeDtypeStruct((bn, dd), jnp.float32),
        scratch_types=[
            pltpu.VMEM((nch, chunk), jnp.int32),
            pltpu.VMEM((bpw, dd), jnp.float32),
            pltpu.SemaphoreType.DMA,
        ],
    )
    def gk(table_hbm, idx_hbm, out_hbm, idx_v, rows_v, sem):
        wid = lax.axis_index("s") * nc + lax.axis_index("c")
        base = wid * bpw
        pltpu.sync_copy(idx_hbm.at[wid], idx_v)
        cps = [pltpu.async_copy(table_hbm.at[idx_v.at[j]],
                                rows_v.at[pl.ds(j * chunk, chunk)], sem)
               for j in range(nch)]
        for j, c in enumerate(cps):
            c.wait()
            pltpu.sync_copy(rows_v.at[pl.ds(j * chunk, chunk)],
                            out_hbm.at[pl.ds(base + j * chunk, chunk)])

    return gk(table, idx3)


def _pad_heads(w, nh):
    w3 = w.reshape(D, nh, HD)
    w3 = jnp.pad(w3, ((0, 0), (0, 0), (0, HP - HD)))
    return w3.reshape(D, nh * HP)


def _rope_tables():
    inv = 1.0 / (THETA ** (jnp.arange(0, HD, 2, dtype=jnp.float32) / HD))
    pos = jnp.arange(S, dtype=jnp.float32)
    fr = jnp.outer(pos, inv)
    cos_e = jnp.repeat(jnp.cos(fr), 2, axis=1)
    sin_b = jnp.sin(fr)
    sin_s = jnp.stack([-sin_b, sin_b], axis=-1).reshape(S, HD)
    return cos_e, sin_s


def kernel(x, qkv_W, out_W, norm1_w, norm2_w, gate_W, W_gate_up, W_down):
    f32 = jnp.float32
    x2 = x.reshape(S, D)
    scale = 1.0 / jnp.sqrt(jnp.asarray(HD, f32))
    cos_e, sin_s = _rope_tables()
    cos_p = jnp.pad(cos_e, ((0, 0), (0, HP - HD)))
    sin_p = jnp.pad(sin_s, ((0, 0), (0, HP - HD)))
    cq = jnp.tile(cos_e * scale, (1, HQ))
    sq = jnp.tile(sin_s * scale, (1, HQ))
    ck = jnp.tile(cos_p, (1, HKV))
    sk = jnp.tile(sin_p, (1, HKV))
    qw = qkv_W[:, :HQ * HD]
    kw = _pad_heads(qkv_W[:, HQ * HD:(HQ + HKV) * HD], HKV)
    vw = _pad_heads(qkv_W[:, (HQ + HKV) * HD:], HKV)
    ow = out_W
    w1r = norm1_w.reshape(1, D)
    w2r = norm2_w.reshape(1, D)

    nr = S // RB
    q, k, v = pl.pallas_call(
        _stage_a_kern,
        grid=(nr,),
        in_specs=[
            pl.BlockSpec((RB, D), lambda r: (r, 0)),
            pl.BlockSpec((1, D), lambda r: (0, 0)),
            pl.BlockSpec((D, D), lambda r: (0, 0)),
            pl.BlockSpec((D, DKP), lambda r: (0, 0)),
            pl.BlockSpec((D, DKP), lambda r: (0, 0)),
            pl.BlockSpec((RB, D), lambda r: (r, 0)),
            pl.BlockSpec((RB, D), lambda r: (r, 0)),
            pl.BlockSpec((RB, DKP), lambda r: (r, 0)),
            pl.BlockSpec((RB, DKP), lambda r: (r, 0)),
        ],
        out_specs=[
            pl.BlockSpec((RB, D), lambda r: (r, 0)),
            pl.BlockSpec((RB, DKP), lambda r: (r, 0)),
            pl.BlockSpec((RB, DKP), lambda r: (r, 0)),
        ],
        out_shape=[
            jax.ShapeDtypeStruct((S, D), f32),
            jax.ShapeDtypeStruct((S, DKP), f32),
            jax.ShapeDtypeStruct((S, DKP), f32),
        ],
    )(x2, w1r, qw, kw, vw, cq, sq, ck, sk)

    ctx = None
    for half in range(4):
        ncols = (half + 1) * 2 * RB
        in_specs = [
            pl.BlockSpec((RB, 4 * HD), lambda g, r, h=half: (r + 2 * h, g)),
            pl.BlockSpec((ncols, HP), lambda g, r: (0, g)),
            pl.BlockSpec((ncols, HP), lambda g, r: (0, g)),
        ]
        ins = [q, k, v]
        aliases = {}
        if ctx is not None:
            in_specs.append(pl.BlockSpec(memory_space=pltpu.MemorySpace.HBM))
            ins.append(ctx)
            aliases = {3: 0}
        ctx = pl.pallas_call(
            functools.partial(_stage_b_kern, qb_off=2 * half, ncols=ncols),
            grid=(HKV, 2),
            in_specs=in_specs,
            out_specs=pl.BlockSpec((RB, 4 * HD),
                                   lambda g, r, h=half: (r + 2 * h, g)),
            out_shape=jax.ShapeDtypeStruct((S, D), f32),
            input_output_aliases=aliases,
        )(*ins)

    h, h2, ti8, tw8 = pl.pallas_call(
        _stage_c_kern,
        grid=(nr,),
        in_specs=[
            pl.BlockSpec((RB, D), lambda r: (r, 0)),
            pl.BlockSpec((RB, D), lambda r: (r, 0)),
            pl.BlockSpec((D, D), lambda r: (0, 0)),
            pl.BlockSpec((1, D), lambda r: (0, 0)),
            pl.BlockSpec((D, E), lambda r: (0, 0)),
        ],
        out_specs=[
            pl.BlockSpec((RB, D), lambda r: (r, 0)),
            pl.BlockSpec((RB, D), lambda r: (r, 0)),
            pl.BlockSpec((RB, E), lambda r: (r, 0)),
            pl.BlockSpec((RB, E), lambda r: (r, 0)),
        ],
        out_shape=[
            jax.ShapeDtypeStruct((S, D), f32),
            jax.ShapeDtypeStruct((S, D), f32),
            jax.ShapeDtypeStruct((S, E), jnp.int32),
            jax.ShapeDtypeStruct((S, E), f32),
        ],
    )(ctx, x2, ow, w2r, gate_W)

    e = ti8[:, :TOPK].reshape(-1)
    perm = jnp.argsort(e, stable=True).astype(jnp.int32)
    es = e[perm]
    counts = jnp.zeros((E,), jnp.int32).at[e].add(1)
    start = jnp.concatenate([jnp.zeros((1,), jnp.int32),
                             jnp.cumsum(counts)[:-1].astype(jnp.int32)])
    pcounts = ((counts + BLK - 1) // BLK) * BLK
    pstart = jnp.concatenate([jnp.zeros((1,), jnp.int32),
                              jnp.cumsum(pcounts)[:-1].astype(jnp.int32)])
    ar = jnp.arange(S * TOPK, dtype=jnp.int32)
    dest = pstart[es] + ar - start[es]
    row_token = jnp.zeros((R,), jnp.int32).at[dest].set(perm // TOPK)
    slot_pos = jnp.zeros((S * TOPK,), jnp.int32).at[perm].set(dest)
    block_expert = jnp.clip(
        jnp.searchsorted(pstart, jnp.arange(NB, dtype=jnp.int32) * BLK,
                         side='right').astype(jnp.int32) - 1, 0, E - 1)
    total_used = pstart[E - 1] + pcounts[E - 1]
    block_live = (jnp.arange(NB, dtype=jnp.int32) * BLK
                  < total_used).astype(jnp.int32)

    xs = _sc_gather(h2, row_token, 40)

    y = pl.pallas_call(
        _stage_d_kern,
        grid_spec=pltpu.PrefetchScalarGridSpec(
            num_scalar_prefetch=2,
            grid=(NB,),
            in_specs=[
                pl.BlockSpec((BLK, D), lambda b, be, bl: (b, 0)),
                pl.BlockSpec((1, D, 2 * HEXP),
                             lambda b, be, bl: (be[b], 0, 0)),
                pl.BlockSpec((1, HEXP, D), lambda b, be, bl: (be[b], 0, 0)),
            ],
            out_specs=pl.BlockSpec((BLK, D), lambda b, be, bl: (b, 0)),
        ),
        out_shape=jax.ShapeDtypeStruct((R, D), f32),
    )(block_expert, block_live, xs, W_gate_up, W_down)

    sp2 = slot_pos.reshape(S, TOPK).T.reshape(-1)
    yg = _sc_gather(y, sp2, 64)

    out = pl.pallas_call(
        _stage_e_kern,
        grid=(nr,),
        in_specs=[
            pl.BlockSpec((RB, D), lambda r: (r, 0)),
            pl.BlockSpec((RB, E), lambda r: (r, 0)),
            pl.BlockSpec((RB, D), lambda r: (r, 0)),
            pl.BlockSpec((RB, D), lambda r: (nr + r, 0)),
        ],
        out_specs=pl.BlockSpec((RB, D), lambda r: (r, 0)),
        out_shape=jax.ShapeDtypeStruct((S, D), f32),
    )(h, tw8, yg, yg)

    return out.reshape(1, S, D)


# speedup vs baseline: 1.0031x; 1.0031x over previous
import functools

import jax
import jax.numpy as jnp
from jax import lax
from jax.experimental import pallas as pl
from jax.experimental.pallas import tpu as pltpu
from jax.experimental.pallas import tpu_sc as plsc

S, D = 2048, 768
HQ, HKV, HD = 12, 3, 64
E, TOPK, HEXP = 8, 2, 1536
THETA = 10000.0
EPS = 1e-6
HP = 128
DKP = HKV * HP
RB = 256
BLK = 128
R = S * TOPK + E * BLK
NB = R // BLK
NEG = -1e30


def _rope(t, c, s):
    lane = jax.lax.broadcasted_iota(jnp.int32, t.shape, 1)
    sw = jnp.where(lane % 2 == 0,
                   jnp.roll(t, -1, axis=1),
                   jnp.roll(t, 1, axis=1))
    return t * c + sw * s


def _stage_a_kern(x_ref, w1_ref, qw_ref, kw_ref, vw_ref,
                  cq_ref, sq_ref, ck_ref, sk_ref,
                  q_ref, k_ref, v_ref):
    xb = x_ref[...]
    ms = jnp.mean(xb * xb, axis=1, keepdims=True)
    xn = (xb / jnp.sqrt(ms + EPS)) * w1_ref[...]
    q = jnp.dot(xn, qw_ref[...], preferred_element_type=jnp.float32)
    k = jnp.dot(xn, kw_ref[...], preferred_element_type=jnp.float32)
    v = jnp.dot(xn, vw_ref[...], preferred_element_type=jnp.float32)
    q_ref[...] = _rope(q, cq_ref[...], sq_ref[...])
    k_ref[...] = _rope(k, ck_ref[...], sk_ref[...])
    v_ref[...] = v


def _stage_b_kern(q_ref, k_ref, v_ref, *refs, qb_off, ncols):
    o_ref = refs[-1]
    qb = pl.program_id(1) + qb_off
    kh = k_ref[:, :HD]
    vh = v_ref[:, :HD]
    row = jax.lax.broadcasted_iota(jnp.int32, (RB, ncols), 0) + qb * RB
    col = jax.lax.broadcasted_iota(jnp.int32, (RB, ncols), 1)
    mask = row >= col
    ctxs = []
    for j in range(4):
        qh = q_ref[:, j * HD:(j + 1) * HD]
        sc = jax.lax.dot_general(qh, kh, (((1,), (1,)), ((), ())),
                                 preferred_element_type=jnp.float32)
        sc = jnp.where(mask, sc, NEG)
        m = jnp.max(sc, axis=1, keepdims=True)
        p = jnp.exp(sc - m)
        p = jnp.where(mask, p, 0.0)
        p = p / jnp.sum(p, axis=1, keepdims=True)
        ctxs.append(jnp.dot(p, vh, preferred_element_type=jnp.float32))
    o_ref[:, 0:2 * HD] = jnp.concatenate(ctxs[0:2], axis=1)
    o_ref[:, 2 * HD:4 * HD] = jnp.concatenate(ctxs[2:4], axis=1)


def _stage_c_kern(ctx_ref, x_ref, ow_ref, w2_ref, gw_ref,
                  h_ref, h2_ref, ti_ref, tw_ref):
    h = x_ref[...] + jnp.dot(ctx_ref[...], ow_ref[...],
                             preferred_element_type=jnp.float32)
    h_ref[...] = h
    ms = jnp.mean(h * h, axis=1, keepdims=True)
    h2 = (h / jnp.sqrt(ms + EPS)) * w2_ref[...]
    h2_ref[...] = h2
    logits = jnp.dot(h2, gw_ref[...], preferred_element_type=jnp.float32)
    lane = jax.lax.broadcasted_iota(jnp.int32, (RB, E), 1)
    m1 = jnp.max(logits, axis=1, keepdims=True)
    i1 = jnp.min(jnp.where(logits == m1, lane, E), axis=1, keepdims=True)
    l2 = jnp.where(lane == i1, NEG, logits)
    m2 = jnp.max(l2, axis=1, keepdims=True)
    i2 = jnp.min(jnp.where(l2 == m2, lane, E), axis=1, keepdims=True)
    w1 = 1.0 / (1.0 + jnp.exp(m2 - m1))
    ti_ref[...] = jnp.where(lane == 0, i1, jnp.where(lane == 1, i2, 0))
    tw_ref[...] = jnp.where(lane == 0, w1, jnp.where(lane == 1, 1.0 - w1, 0.0))


def _stage_d_kern(be_ref, bl_ref, xs_ref, wgu_ref, wd_ref, y_ref):
    @pl.when(bl_ref[pl.program_id(0)] != 0)
    def _():
        gu = jnp.dot(xs_ref[...], wgu_ref[0],
                     preferred_element_type=jnp.float32)
        g = gu[:, :HEXP]
        u = gu[:, HEXP:]
        act = (g / (1.0 + jnp.exp(-g))) * u
        y_ref[...] = jnp.dot(act, wd_ref[0],
                             preferred_element_type=jnp.float32)


def _stage_e_kern(h_ref, tw_ref, y0_ref, y1_ref, o_ref):
    w0 = tw_ref[:, 0:1]
    w1 = tw_ref[:, 1:2]
    o_ref[...] = h_ref[...] + w0 * y0_ref[...] + w1 * y1_ref[...]


def _sc_gather(table, idx, chunk):
    bn = idx.shape[0]
    dd = table.shape[1]
    info = plsc.get_sparse_core_info()
    nc, ns = info.num_cores, info.num_subcores
    nw = nc * ns
    bpw = bn // nw
    nch = bpw // chunk
    idx3 = idx.reshape(nw, nch, chunk)

    @functools.partial(
        pl.kernel,
        mesh=plsc.VectorSubcoreMesh(core_axis_name="c", subcore_axis_name="s"),
        out_type=jax.ShapeDtypeStruct((bn, dd), jnp.float32),
        scratch_types=[
            pltpu.VMEM((nch, chunk), jnp.int32),
            pltpu.VMEM((bpw, dd), jnp.float32),
            pltpu.SemaphoreType.DMA,
        ],
    )
    def gk(table_hbm, idx_hbm, out_hbm, idx_v, rows_v, sem):
        wid = lax.axis_index("s") * nc + lax.axis_index("c")
        base = wid * bpw
        pltpu.sync_copy(idx_hbm.at[wid], idx_v)
        cps = [pltpu.async_copy(table_hbm.at[idx_v.at[j]],
                                rows_v.at[pl.ds(j * chunk, chunk)], sem)
               for j in range(nch)]
        for j, c in enumerate(cps):
            c.wait()
            pltpu.sync_copy(rows_v.at[pl.ds(j * chunk, chunk)],
                            out_hbm.at[pl.ds(base + j * chunk, chunk)])

    return gk(table, idx3)


def _pad_heads(w, nh):
    w3 = w.reshape(D, nh, HD)
    w3 = jnp.pad(w3, ((0, 0), (0, 0), (0, HP - HD)))
    return w3.reshape(D, nh * HP)


def _rope_tables():
    inv = 1.0 / (THETA ** (jnp.arange(0, HD, 2, dtype=jnp.float32) / HD))
    pos = jnp.arange(S, dtype=jnp.float32)
    fr = jnp.outer(pos, inv)
    cos_e = jnp.repeat(jnp.cos(fr), 2, axis=1)
    sin_b = jnp.sin(fr)
    sin_s = jnp.stack([-sin_b, sin_b], axis=-1).reshape(S, HD)
    return cos_e, sin_s


def kernel(x, qkv_W, out_W, norm1_w, norm2_w, gate_W, W_gate_up, W_down):
    f32 = jnp.float32
    x2 = x.reshape(S, D)
    scale = 1.0 / jnp.sqrt(jnp.asarray(HD, f32))
    cos_e, sin_s = _rope_tables()
    cos_p = jnp.pad(cos_e, ((0, 0), (0, HP - HD)))
    sin_p = jnp.pad(sin_s, ((0, 0), (0, HP - HD)))
    cq = jnp.tile(cos_e * scale, (1, HQ))
    sq = jnp.tile(sin_s * scale, (1, HQ))
    ck = jnp.tile(cos_p, (1, HKV))
    sk = jnp.tile(sin_p, (1, HKV))
    qw = qkv_W[:, :HQ * HD]
    kw = _pad_heads(qkv_W[:, HQ * HD:(HQ + HKV) * HD], HKV)
    vw = _pad_heads(qkv_W[:, (HQ + HKV) * HD:], HKV)
    ow = out_W
    w1r = norm1_w.reshape(1, D)
    w2r = norm2_w.reshape(1, D)

    nr = S // RB
    q, k, v = pl.pallas_call(
        _stage_a_kern,
        grid=(nr,),
        in_specs=[
            pl.BlockSpec((RB, D), lambda r: (r, 0)),
            pl.BlockSpec((1, D), lambda r: (0, 0)),
            pl.BlockSpec((D, D), lambda r: (0, 0)),
            pl.BlockSpec((D, DKP), lambda r: (0, 0)),
            pl.BlockSpec((D, DKP), lambda r: (0, 0)),
            pl.BlockSpec((RB, D), lambda r: (r, 0)),
            pl.BlockSpec((RB, D), lambda r: (r, 0)),
            pl.BlockSpec((RB, DKP), lambda r: (r, 0)),
            pl.BlockSpec((RB, DKP), lambda r: (r, 0)),
        ],
        out_specs=[
            pl.BlockSpec((RB, D), lambda r: (r, 0)),
            pl.BlockSpec((RB, DKP), lambda r: (r, 0)),
            pl.BlockSpec((RB, DKP), lambda r: (r, 0)),
        ],
        out_shape=[
            jax.ShapeDtypeStruct((S, D), f32),
            jax.ShapeDtypeStruct((S, DKP), f32),
            jax.ShapeDtypeStruct((S, DKP), f32),
        ],
    )(x2, w1r, qw, kw, vw, cq, sq, ck, sk)

    ctx = None
    for half in range(4):
        ncols = (half + 1) * 2 * RB
        in_specs = [
            pl.BlockSpec((RB, 4 * HD), lambda g, r, h=half: (r + 2 * h, g)),
            pl.BlockSpec((ncols, HP), lambda g, r: (0, g)),
            pl.BlockSpec((ncols, HP), lambda g, r: (0, g)),
        ]
        ins = [q, k, v]
        aliases = {}
        if ctx is not None:
            in_specs.append(pl.BlockSpec(memory_space=pltpu.MemorySpace.HBM))
            ins.append(ctx)
            aliases = {3: 0}
        ctx = pl.pallas_call(
            functools.partial(_stage_b_kern, qb_off=2 * half, ncols=ncols),
            grid=(HKV, 2),
            in_specs=in_specs,
            out_specs=pl.BlockSpec((RB, 4 * HD),
                                   lambda g, r, h=half: (r + 2 * h, g)),
            out_shape=jax.ShapeDtypeStruct((S, D), f32),
            input_output_aliases=aliases,
        )(*ins)

    h, h2, ti8, tw8 = pl.pallas_call(
        _stage_c_kern,
        grid=(nr,),
        in_specs=[
            pl.BlockSpec((RB, D), lambda r: (r, 0)),
            pl.BlockSpec((RB, D), lambda r: (r, 0)),
            pl.BlockSpec((D, D), lambda r: (0, 0)),
            pl.BlockSpec((1, D), lambda r: (0, 0)),
            pl.BlockSpec((D, E), lambda r: (0, 0)),
        ],
        out_specs=[
            pl.BlockSpec((RB, D), lambda r: (r, 0)),
            pl.BlockSpec((RB, D), lambda r: (r, 0)),
            pl.BlockSpec((RB, E), lambda r: (r, 0)),
            pl.BlockSpec((RB, E), lambda r: (r, 0)),
        ],
        out_shape=[
            jax.ShapeDtypeStruct((S, D), f32),
            jax.ShapeDtypeStruct((S, D), f32),
            jax.ShapeDtypeStruct((S, E), jnp.int32),
            jax.ShapeDtypeStruct((S, E), f32),
        ],
    )(ctx, x2, ow, w2r, gate_W)

    e = ti8[:, :TOPK].reshape(-1)
    perm = jnp.argsort(e, stable=True).astype(jnp.int32)
    es = e[perm]
    counts = jnp.zeros((E,), jnp.int32).at[e].add(1)
    start = jnp.concatenate([jnp.zeros((1,), jnp.int32),
                             jnp.cumsum(counts)[:-1].astype(jnp.int32)])
    pcounts = ((counts + BLK - 1) // BLK) * BLK
    pstart = jnp.concatenate([jnp.zeros((1,), jnp.int32),
                              jnp.cumsum(pcounts)[:-1].astype(jnp.int32)])
    ar = jnp.arange(S * TOPK, dtype=jnp.int32)
    dest = pstart[es] + ar - start[es]
    row_token = jnp.zeros((R,), jnp.int32).at[dest].set(perm // TOPK)
    slot_pos = jnp.zeros((S * TOPK,), jnp.int32).at[perm].set(dest)
    block_expert = jnp.clip(
        jnp.searchsorted(pstart, jnp.arange(NB, dtype=jnp.int32) * BLK,
                         side='right').astype(jnp.int32) - 1, 0, E - 1)
    total_used = pstart[E - 1] + pcounts[E - 1]
    block_live = (jnp.arange(NB, dtype=jnp.int32) * BLK
                  < total_used).astype(jnp.int32)

    xs = _sc_gather(h2, row_token, 80)

    y = pl.pallas_call(
        _stage_d_kern,
        grid_spec=pltpu.PrefetchScalarGridSpec(
            num_scalar_prefetch=2,
            grid=(NB,),
            in_specs=[
                pl.BlockSpec((BLK, D), lambda b, be, bl: (b, 0)),
                pl.BlockSpec((1, D, 2 * HEXP),
                             lambda b, be, bl: (be[b], 0, 0)),
                pl.BlockSpec((1, HEXP, D), lambda b, be, bl: (be[b], 0, 0)),
            ],
            out_specs=pl.BlockSpec((BLK, D), lambda b, be, bl: (b, 0)),
        ),
        out_shape=jax.ShapeDtypeStruct((R, D), f32),
    )(block_expert, block_live, xs, W_gate_up, W_down)

    sp2 = slot_pos.reshape(S, TOPK).T.reshape(-1)
    yg = _sc_gather(y, sp2, 64)

    out = pl.pallas_call(
        _stage_e_kern,
        grid=(nr,),
        in_specs=[
            pl.BlockSpec((RB, D), lambda r: (r, 0)),
            pl.BlockSpec((RB, E), lambda r: (r, 0)),
            pl.BlockSpec((RB, D), lambda r: (r, 0)),
            pl.BlockSpec((RB, D), lambda r: (nr + r, 0)),
        ],
        out_specs=pl.BlockSpec((RB, D), lambda r: (r, 0)),
        out_shape=jax.ShapeDtypeStruct((S, D), f32),
    )(h, tw8, yg, yg)

    return out.reshape(1, S, D)
